# baseline (speedup 1.0000x reference)
import functools

import jax
import jax.numpy as jnp
from jax import lax
from jax.experimental import pallas as pl
from jax.experimental.pallas import tpu as pltpu
from jax.experimental.pallas import tpu_sc as plsc

B, N, K, H = 4, 1024, 36, 128
TN = 128
NW = 32
SPLITS = 4
BS = B // SPLITS
EPW = BS * N * K // NW
CHUNK = 128
NCHUNK = EPW // CHUNK
WPB = NW // BS


def _gelu(x):
    return x * (0.5 + 0.5 * lax.erf(x * 0.7071067811865476))


def _ln(x, g, b):
    m = jnp.mean(x, axis=-1, keepdims=True)
    v = jnp.mean((x - m) ** 2, axis=-1, keepdims=True)
    return (x - m) * lax.rsqrt(v + 1e-5) * g + b


def _proj_body(hv_ref, w_ref, p_ref):
    p_ref[...] = jnp.dot(hv_ref[...], w_ref[...],
                         preferred_element_type=jnp.float32)


def _project(h_V_flat, W1c):
    return pl.pallas_call(
        _proj_body,
        out_shape=jax.ShapeDtypeStruct((B * N, H), jnp.float32),
    )(h_V_flat, W1c)


def _gather(P_flat, idx_resh, base_b):
    mesh = plsc.VectorSubcoreMesh(core_axis_name="c", subcore_axis_name="s")

    @functools.partial(
        pl.kernel, mesh=mesh,
        out_type=jax.ShapeDtypeStruct((BS * K * N, H), jnp.float32),
        scratch_types=[
            pltpu.VMEM((NCHUNK, CHUNK), jnp.int32),
            pltpu.VMEM((CHUNK, H), jnp.float32),
            pltpu.VMEM((CHUNK, H), jnp.float32),
            pltpu.SemaphoreType.DMA,
            pltpu.SemaphoreType.DMA,
        ],
    )
    def gk(p_hbm, idx_hbm, out_hbm, idx_v, rows0, rows1, sem0, sem1):
        wid = lax.axis_index("s") * 2 + lax.axis_index("c")
        base = wid * EPW
        boff = (base_b + wid // WPB) * N
        pltpu.sync_copy(idx_hbm.at[wid], idx_v)

        def add_off(i, _):
            def add16(j, _):
                sl = (i, pl.ds(j * 16, 16))
                idx_v[sl] = idx_v[sl] + boff
                return 0
            return lax.fori_loop(0, CHUNK // 16, add16, 0)
        lax.fori_loop(0, NCHUNK, add_off, 0)

        pltpu.async_copy(p_hbm.at[idx_v.at[0]], rows0, sem0)

        def chunk(c, _):
            even = c % 2 == 0
            @pl.when(jnp.logical_and(even, c + 1 < NCHUNK))
            def _():
                pltpu.make_async_copy(p_hbm.at[idx_v.at[c + 1]], rows1,
                                      sem1).start()
            @pl.when(jnp.logical_and(jnp.logical_not(even), c + 1 < NCHUNK))
            def _():
                pltpu.make_async_copy(p_hbm.at[idx_v.at[c + 1]], rows0,
                                      sem0).start()
            @pl.when(even)
            def _():
                pltpu.make_async_copy(p_hbm.at[idx_v.at[c]], rows0,
                                      sem0).wait()
                pltpu.sync_copy(rows0,
                                out_hbm.at[pl.ds(base + c * CHUNK, CHUNK)])
            @pl.when(jnp.logical_not(even))
            def _():
                pltpu.make_async_copy(p_hbm.at[idx_v.at[c]], rows1,
                                      sem1).wait()
                pltpu.sync_copy(rows1,
                                out_hbm.at[pl.ds(base + c * CHUNK, CHUNK)])
            return 0
        lax.fori_loop(0, NCHUNK, chunk, 0)

    return gk(P_flat, idx_resh)


def kernel(h_V, h_E, E_idx, mask_V, mask_attend, W1, b1, W2, b2, W3, b3,
           Win, bi, Wout, bo, g1, be1, g2, be2):
    bf = jnp.bfloat16
    W1a, W1b, W1c = W1[:H], W1[H:2 * H], W1[2 * H:]

    hEt = jnp.transpose(h_E, (0, 2, 1, 3))
    Eit = jnp.transpose(E_idx, (0, 2, 1))
    idx_all = Eit.reshape(SPLITS, NW, NCHUNK, CHUNK)
    mat = jnp.transpose(mask_attend, (0, 2, 1))
    maf2 = (mat.reshape(B, K, N // TN, TN).transpose(0, 2, 1, 3)
            .reshape(B, N // TN, 1, K * TN).astype(bf))

    jcol = jnp.arange(K * TN, dtype=jnp.int32) % TN
    A2 = (jnp.arange(TN, dtype=jnp.int32)[:, None]
          == jcol[None, :]).astype(bf)
    R2 = A2.T

    P = _project(h_V.reshape(B * N, H), W1c)

    def body(hv_ref, he_ref, g_ref, ma_ref, maf_ref, mv_ref, a_ref, r_ref,
             w1a_ref, b1_ref, w1b_ref, w2_ref, b2_ref, w3_ref, b3_ref,
             win_ref, bi_ref, wout_ref, bo_ref,
             g1_ref, be1_ref, g2_ref, be2_ref, out_ref):
        hv = hv_ref[0]
        he = he_ref[0].reshape(K * TN, H).astype(bf)
        gg = g_ref[...].reshape(K * TN, H)
        ma = ma_ref[0]
        maf = maf_ref[0, 0]

        q = jnp.dot(hv, w1a_ref[...],
                    preferred_element_type=jnp.float32) + b1_ref[...]
        qrep = jnp.dot(r_ref[...], q.astype(bf),
                       preferred_element_type=jnp.float32)
        pre1 = jnp.dot(he, w1b_ref[...],
                       preferred_element_type=jnp.float32) + gg + qrep
        m1 = _gelu(pre1).astype(bf)
        m2 = _gelu(jnp.dot(m1, w2_ref[...],
                           preferred_element_type=jnp.float32) + b2_ref[...])
        am = a_ref[...] * maf
        s = jnp.dot(am, m2.astype(bf),
                    preferred_element_type=jnp.float32)
        cnt = jnp.sum(ma, axis=1, keepdims=True)
        dh = (jnp.dot(s, w3_ref[...], preferred_element_type=jnp.float32)
              + cnt * b3_ref[...]) * (1.0 / 36.0)
        x = _ln(hv + dh, g1_ref[...], be1_ref[...])
        ffn = jnp.dot(_gelu(jnp.dot(x.astype(bf), win_ref[...],
                                    preferred_element_type=jnp.float32)
                            + bi_ref[...]).astype(bf),
                      wout_ref[...],
                      preferred_element_type=jnp.float32) + bo_ref[...]
        y = _ln(x + ffn, g2_ref[...], be2_ref[...])
        out_ref[0] = y * mv_ref[0, 0, 0][:, None]

    w_spec2 = lambda shp: pl.BlockSpec(shp, lambda b, t: (0, 0))
    outs = []
    for s in range(SPLITS):
        G = _gather(P, idx_all[s], s * BS).reshape(BS * K, N, H)
        out_s = pl.pallas_call(
            body,
            grid=(BS, N // TN),
            in_specs=[
                pl.BlockSpec((1, TN, H),
                             lambda b, t, s=s: (s * BS + b, t, 0)),
                pl.BlockSpec((1, K, TN, H),
                             lambda b, t, s=s: (s * BS + b, 0, t, 0)),
                pl.BlockSpec((K, TN, H), lambda b, t: (b, t, 0)),
                pl.BlockSpec((1, TN, K),
                             lambda b, t, s=s: (s * BS + b, t, 0)),
                pl.BlockSpec((1, 1, 1, K * TN),
                             lambda b, t, s=s: (s * BS + b, t, 0, 0)),
                pl.BlockSpec((1, 1, 1, TN),
                             lambda b, t, s=s: (s * BS + b, t, 0, 0)),
                w_spec2((TN, K * TN)),
                w_spec2((K * TN, TN)),
                w_spec2((H, H)),
                w_spec2((1, H)),
                w_spec2((H, H)),
                w_spec2((H, H)),
                w_spec2((1, H)),
                w_spec2((H, H)),
                w_spec2((1, H)),
                w_spec2((H, 4 * H)),
                w_spec2((1, 4 * H)),
                w_spec2((4 * H, H)),
                w_spec2((1, H)),
                w_spec2((1, H)),
                w_spec2((1, H)),
                w_spec2((1, H)),
                w_spec2((1, H)),
            ],
            out_specs=pl.BlockSpec((1, TN, H), lambda b, t: (b, t, 0)),
            out_shape=jax.ShapeDtypeStruct((BS, N, H), jnp.float32),
        )(h_V, hEt, G, mask_attend, maf2,
          mask_V.reshape(B, N // TN, 1, TN),
          A2, R2,
          W1a, b1.reshape(1, H), W1b.astype(bf),
          W2.astype(bf), b2.reshape(1, H), W3,
          b3.reshape(1, H), Win.astype(bf),
          bi.reshape(1, 4 * H), Wout.astype(bf), bo.reshape(1, H),
          g1.reshape(1, H), be1.reshape(1, H), g2.reshape(1, H),
          be2.reshape(1, H))
        outs.append(out_s)
    return jnp.concatenate(outs, axis=0)


# speedup vs baseline: 1.0603x; 1.0603x over previous
import functools

import jax
import jax.numpy as jnp
from jax import lax
from jax.experimental import pallas as pl
from jax.experimental.pallas import tpu as pltpu
from jax.experimental.pallas import tpu_sc as plsc

B, N, K, H = 4, 1024, 36, 128
TN = 128
NW = 32
SPLITS = 2
BS = B // SPLITS
EPW = BS * N * K // NW
CHUNK = 128
NCHUNK = EPW // CHUNK
WPB = NW // BS


def _gelu(x):
    return x * (0.5 + 0.5 * lax.erf(x * 0.7071067811865476))


def _ln(x, g, b):
    m = jnp.mean(x, axis=-1, keepdims=True)
    v = jnp.mean((x - m) ** 2, axis=-1, keepdims=True)
    return (x - m) * lax.rsqrt(v + 1e-5) * g + b


def _proj_body(hv_ref, w_ref, p_ref):
    p_ref[...] = jnp.dot(hv_ref[...], w_ref[...],
                         preferred_element_type=jnp.float32)


def _project(h_V_flat, W1c):
    return pl.pallas_call(
        _proj_body,
        out_shape=jax.ShapeDtypeStruct((B * N, H), jnp.float32),
    )(h_V_flat, W1c)


def _gather(P_flat, idx_resh, base_b):
    mesh = plsc.VectorSubcoreMesh(core_axis_name="c", subcore_axis_name="s")

    @functools.partial(
        pl.kernel, mesh=mesh,
        out_type=jax.ShapeDtypeStruct((BS * K * N, H), jnp.float32),
        scratch_types=[
            pltpu.VMEM((NCHUNK, CHUNK), jnp.int32),
            pltpu.VMEM((CHUNK, H), jnp.float32),
            pltpu.VMEM((CHUNK, H), jnp.float32),
            pltpu.SemaphoreType.DMA,
            pltpu.SemaphoreType.DMA,
        ],
    )
    def gk(p_hbm, idx_hbm, out_hbm, idx_v, rows0, rows1, sem0, sem1):
        wid = lax.axis_index("s") * 2 + lax.axis_index("c")
        base = wid * EPW
        boff = (base_b + wid // WPB) * N
        pltpu.sync_copy(idx_hbm.at[wid], idx_v)

        def add_off(i, _):
            def add16(j, _):
                sl = (i, pl.ds(j * 16, 16))
                idx_v[sl] = idx_v[sl] + boff
                return 0
            return lax.fori_loop(0, CHUNK // 16, add16, 0)
        lax.fori_loop(0, NCHUNK, add_off, 0)

        pltpu.async_copy(p_hbm.at[idx_v.at[0]], rows0, sem0)

        def chunk(c, _):
            even = c % 2 == 0
            @pl.when(jnp.logical_and(even, c + 1 < NCHUNK))
            def _():
                pltpu.make_async_copy(p_hbm.at[idx_v.at[c + 1]], rows1,
                                      sem1).start()
            @pl.when(jnp.logical_and(jnp.logical_not(even), c + 1 < NCHUNK))
            def _():
                pltpu.make_async_copy(p_hbm.at[idx_v.at[c + 1]], rows0,
                                      sem0).start()
            @pl.when(even)
            def _():
                pltpu.make_async_copy(p_hbm.at[idx_v.at[c]], rows0,
                                      sem0).wait()
                pltpu.sync_copy(rows0,
                                out_hbm.at[pl.ds(base + c * CHUNK, CHUNK)])
            @pl.when(jnp.logical_not(even))
            def _():
                pltpu.make_async_copy(p_hbm.at[idx_v.at[c]], rows1,
                                      sem1).wait()
                pltpu.sync_copy(rows1,
                                out_hbm.at[pl.ds(base + c * CHUNK, CHUNK)])
            return 0
        lax.fori_loop(0, NCHUNK, chunk, 0)

    return gk(P_flat, idx_resh)


def kernel(h_V, h_E, E_idx, mask_V, mask_attend, W1, b1, W2, b2, W3, b3,
           Win, bi, Wout, bo, g1, be1, g2, be2):
    bf = jnp.bfloat16
    W1a, W1b, W1c = W1[:H], W1[H:2 * H], W1[2 * H:]

    hEt = jnp.transpose(h_E, (0, 2, 1, 3))
    Eit = jnp.transpose(E_idx, (0, 2, 1))
    idx_all = Eit.reshape(SPLITS, NW, NCHUNK, CHUNK)
    mat = jnp.transpose(mask_attend, (0, 2, 1))
    maf2 = (mat.reshape(B, K, N // TN, TN).transpose(0, 2, 1, 3)
            .reshape(B, N // TN, 1, K * TN).astype(bf))

    jcol = jnp.arange(K * TN, dtype=jnp.int32) % TN
    A2 = (jnp.arange(TN, dtype=jnp.int32)[:, None]
          == jcol[None, :]).astype(bf)
    R2 = A2.T

    P = _project(h_V.reshape(B * N, H), W1c)

    def body(hv_ref, he_ref, g_ref, ma_ref, maf_ref, mv_ref, a_ref, r_ref,
             w1a_ref, b1_ref, w1b_ref, w2_ref, b2_ref, w3_ref, b3_ref,
             win_ref, bi_ref, wout_ref, bo_ref,
             g1_ref, be1_ref, g2_ref, be2_ref, out_ref):
        hv = hv_ref[0]
        he = he_ref[0].reshape(K * TN, H).astype(bf)
        gg = g_ref[...].reshape(K * TN, H)
        ma = ma_ref[0]
        maf = maf_ref[0, 0]

        q = jnp.dot(hv, w1a_ref[...],
                    preferred_element_type=jnp.float32) + b1_ref[...]
        qrep = jnp.dot(r_ref[...], q.astype(bf),
                       preferred_element_type=jnp.float32)
        pre1 = jnp.dot(he, w1b_ref[...],
                       preferred_element_type=jnp.float32) + gg + qrep
        m1 = _gelu(pre1).astype(bf)
        m2 = _gelu(jnp.dot(m1, w2_ref[...],
                           preferred_element_type=jnp.float32) + b2_ref[...])
        am = a_ref[...] * maf
        s = jnp.dot(am, m2.astype(bf),
                    preferred_element_type=jnp.float32)
        cnt = jnp.sum(ma, axis=1, keepdims=True)
        dh = (jnp.dot(s, w3_ref[...], preferred_element_type=jnp.float32)
              + cnt * b3_ref[...]) * (1.0 / 36.0)
        x = _ln(hv + dh, g1_ref[...], be1_ref[...])
        ffn = jnp.dot(_gelu(jnp.dot(x.astype(bf), win_ref[...],
                                    preferred_element_type=jnp.float32)
                            + bi_ref[...]).astype(bf),
                      wout_ref[...],
                      preferred_element_type=jnp.float32) + bo_ref[...]
        y = _ln(x + ffn, g2_ref[...], be2_ref[...])
        out_ref[0] = y * mv_ref[0, 0, 0][:, None]

    w_spec2 = lambda shp: pl.BlockSpec(shp, lambda b, t: (0, 0))
    outs = []
    for s in range(SPLITS):
        G = _gather(P, idx_all[s], s * BS).reshape(BS * K, N, H)
        out_s = pl.pallas_call(
            body,
            grid=(BS, N // TN),
            in_specs=[
                pl.BlockSpec((1, TN, H),
                             lambda b, t, s=s: (s * BS + b, t, 0)),
                pl.BlockSpec((1, K, TN, H),
                             lambda b, t, s=s: (s * BS + b, 0, t, 0)),
                pl.BlockSpec((K, TN, H), lambda b, t: (b, t, 0)),
                pl.BlockSpec((1, TN, K),
                             lambda b, t, s=s: (s * BS + b, t, 0)),
                pl.BlockSpec((1, 1, 1, K * TN),
                             lambda b, t, s=s: (s * BS + b, t, 0, 0)),
                pl.BlockSpec((1, 1, 1, TN),
                             lambda b, t, s=s: (s * BS + b, t, 0, 0)),
                w_spec2((TN, K * TN)),
                w_spec2((K * TN, TN)),
                w_spec2((H, H)),
                w_spec2((1, H)),
                w_spec2((H, H)),
                w_spec2((H, H)),
                w_spec2((1, H)),
                w_spec2((H, H)),
                w_spec2((1, H)),
                w_spec2((H, 4 * H)),
                w_spec2((1, 4 * H)),
                w_spec2((4 * H, H)),
                w_spec2((1, H)),
                w_spec2((1, H)),
                w_spec2((1, H)),
                w_spec2((1, H)),
                w_spec2((1, H)),
            ],
            out_specs=pl.BlockSpec((1, TN, H), lambda b, t: (b, t, 0)),
            out_shape=jax.ShapeDtypeStruct((BS, N, H), jnp.float32),
        )(h_V, hEt, G, mask_attend, maf2,
          mask_V.reshape(B, N // TN, 1, TN),
          A2, R2,
          W1a, b1.reshape(1, H), W1b.astype(bf),
          W2.astype(bf), b2.reshape(1, H), W3,
          b3.reshape(1, H), Win.astype(bf),
          bi.reshape(1, 4 * H), Wout.astype(bf), bo.reshape(1, H),
          g1.reshape(1, H), be1.reshape(1, H), g2.reshape(1, H),
          be2.reshape(1, H))
        outs.append(out_s)
    return jnp.concatenate(outs, axis=0)
